# baseline (speedup 1.0000x reference)
import functools

import jax
import jax.numpy as jnp
from jax import lax
from jax.experimental import pallas as pl
from jax.experimental.pallas import tpu as pltpu
from jax.experimental.pallas import tpu_sc as plsc

N = 4194304
NC, NS, L = 2, 16, 16
NW = NC * NS
SHARD = N // NW
CHUNK = 8192
NCHUNK = SHARD // CHUNK
VPC = CHUNK // L
NBIN = 65536
HROWS = NBIN // 128
BR = 2048
NROWS = N // 128
NBLK = NROWS // BR
BIG = 2147483647
SIGN = -2147483648

@functools.cache
def _mesh():
    return plsc.VectorSubcoreMesh(
        core_axis_name="c", subcore_axis_name="s", num_cores=NC, num_subcores=NS
    )


def _kds(g):
    bits = lax.bitcast_convert_type(g, jnp.int32)
    return jnp.where(bits < 0, bits ^ SIGN, ~bits)


def _cumsum_axis1(x):
    for s in (1, 2, 4, 8, 16, 32, 64):
        x = x + jnp.concatenate(
            [jnp.zeros((x.shape[0], s), x.dtype), x[:, :-s]], axis=1)
    return x


def _cumsum_axis0(x):
    s = 1
    while s < x.shape[0]:
        x = x + jnp.concatenate(
            [jnp.zeros((s, x.shape[1]), x.dtype), x[:-s, :]], axis=0)
        s *= 2
    return x


def _cumsum_rowmajor(x):
    cl = _cumsum_axis1(x)
    rs = cl[:, 127:128]
    rp = _cumsum_axis0(rs) - rs
    return cl + rp


def _bin2d(rows):
    r = lax.broadcasted_iota(jnp.int32, (rows, 128), 0)
    c = lax.broadcasted_iota(jnp.int32, (rows, 128), 1)
    return r * 128 + c


UNROLL = 16


def _hist1_body(grad_hbm, w_hbm, zeros_hbm, h1_hbm, minv_hbm, mini_hbm,
                gbuf0, gbuf1, wbuf0, wbuf1, hist, vf, vi, sem0, sem1):
    cid = lax.axis_index("c")
    sid = lax.axis_index("s")
    wid = sid * NC + cid
    base = wid * SHARD
    gb = (gbuf0, gbuf1)
    wb = (wbuf0, wbuf1)
    sems = (sem0, sem1)
    zc = pltpu.async_copy(zeros_hbm, hist, sem0)

    def start(k):
        slot = k % 2
        cg = pltpu.async_copy(
            grad_hbm.at[pl.ds(base + k * CHUNK, CHUNK)], gb[slot], sems[slot])
        cw = pltpu.async_copy(
            w_hbm.at[pl.ds(base + k * CHUNK, CHUNK)], wb[slot], sems[slot])
        return cg, cw

    zc.wait()
    pend = start(0)
    lane = lax.iota(jnp.int32, L)
    mn = jnp.full((L,), jnp.inf, jnp.float32)
    mi = jnp.zeros((L,), jnp.int32)
    for k in range(NCHUNK):
        slot = k % 2
        cg, cw = pend
        cg.wait()
        cw.wait()
        if k + 1 < NCHUNK:
            pend = start(k + 1)
        gbuf, wbuf = gb[slot], wb[slot]
        cbase = base + k * CHUNK

        @plsc.parallel_loop(0, VPC, 1, unroll=UNROLL, carry=(mn, mi))
        def mnmi(j, c, gbuf=gbuf, wbuf=wbuf, cbase=cbase):
            mn0, mi0 = c
            off = j * L
            g = gbuf[pl.ds(off, L)]
            wv = wbuf[pl.ds(off, L)]
            kds = _kds(g)
            bin1 = (kds >> 16) + 32768
            plsc.addupdate_scatter(hist, [bin1 >> 7, bin1 & 127], wv)
            m = g < mn0
            gi = lane + (cbase + off)
            return (jnp.where(m, g, mn0), jnp.where(m, gi, mi0))

        mn, mi = mnmi

    vf[...] = mn
    vi[...] = mi
    pltpu.sync_copy(hist, h1_hbm.at[wid])
    pltpu.sync_copy(vf, minv_hbm.at[wid])
    pltpu.sync_copy(vi, mini_hbm.at[wid])


@functools.cache
def _hist1():
    return pl.kernel(
        _hist1_body,
        out_type=[
            jax.ShapeDtypeStruct((NW, HROWS, 128), jnp.float32),
            jax.ShapeDtypeStruct((NW, L), jnp.float32),
            jax.ShapeDtypeStruct((NW, L), jnp.int32),
        ],
        mesh=_mesh(),
        scratch_types=[
            pltpu.VMEM((CHUNK,), jnp.float32),
            pltpu.VMEM((CHUNK,), jnp.float32),
            pltpu.VMEM((CHUNK,), jnp.float32),
            pltpu.VMEM((CHUNK,), jnp.float32),
            pltpu.VMEM((HROWS, 128), jnp.float32),
            pltpu.VMEM((L,), jnp.float32),
            pltpu.VMEM((L,), jnp.int32),
            pltpu.SemaphoreType.DMA,
            pltpu.SemaphoreType.DMA,
        ],
        compiler_params=pltpu.CompilerParams(needs_layout_passes=False),
    )


def _scan1_body(h1_ref, minv_ref, mini_ref, eps_ref, oi_ref, of_ref, ob_ref):
    eps = eps_ref[0, 0]
    H = jnp.sum(h1_ref[...], axis=0)
    mv = minv_ref[...]
    gmin = jnp.min(mv)
    imin = jnp.min(jnp.where(mv == gmin, mini_ref[...], BIG))
    kmin = jnp.max(_kds(mv))
    hbmin = (kmin >> 16) + 32768
    bins = _bin2d(HROWS)
    Hadj = H + jnp.where(bins == hbmin, eps, 0.0)
    cum = _cumsum_rowmajor(Hadj)
    b1 = jnp.min(jnp.where(cum >= eps, bins, BIG))
    hb1 = jnp.sum(jnp.where(bins == b1, Hadj, 0.0))
    e1 = jnp.sum(jnp.where(bins == b1, cum, 0.0)) - hb1
    r1 = jnp.minimum(eps - e1, hb1)
    adj = jnp.where(hbmin == b1, kmin & 65535, jnp.int32(-1))
    oi = jnp.where(bins == 0, b1,
                   jnp.where(bins == 1, imin,
                             jnp.where(bins == 2, adj, 0)))
    oi_ref[...] = oi[:8, :]
    of_ref[...] = jnp.where(bins[:8, :] == 0, r1, 0.0)
    ob_ref[...] = jnp.zeros((NW, L), jnp.int32) + b1


_scan1 = pl.pallas_call(
    _scan1_body,
    in_specs=[
        pl.BlockSpec(memory_space=pltpu.VMEM),
        pl.BlockSpec(memory_space=pltpu.VMEM),
        pl.BlockSpec(memory_space=pltpu.VMEM),
        pl.BlockSpec(memory_space=pltpu.SMEM),
    ],
    out_specs=[
        pl.BlockSpec(memory_space=pltpu.VMEM),
        pl.BlockSpec(memory_space=pltpu.VMEM),
        pl.BlockSpec(memory_space=pltpu.VMEM),
    ],
    out_shape=[
        jax.ShapeDtypeStruct((8, 128), jnp.int32),
        jax.ShapeDtypeStruct((8, 128), jnp.float32),
        jax.ShapeDtypeStruct((NW, L), jnp.int32),
    ],
)


def _hist2_body(grad_hbm, w_hbm, zeros_hbm, b1_hbm, h2_hbm,
                gbuf0, gbuf1, wbuf0, wbuf1, hist, b1buf, sem0, sem1):
    cid = lax.axis_index("c")
    sid = lax.axis_index("s")
    wid = sid * NC + cid
    base = wid * SHARD
    gb = (gbuf0, gbuf1)
    wb = (wbuf0, wbuf1)
    sems = (sem0, sem1)
    pltpu.sync_copy(zeros_hbm, hist)
    pltpu.sync_copy(b1_hbm.at[0], b1buf)
    b1v = b1buf[...]

    def start(k):
        slot = k % 2
        cg = pltpu.async_copy(
            grad_hbm.at[pl.ds(base + k * CHUNK, CHUNK)], gb[slot], sems[slot])
        cw = pltpu.async_copy(
            w_hbm.at[pl.ds(base + k * CHUNK, CHUNK)], wb[slot], sems[slot])
        return cg, cw

    pend = start(0)
    for k in range(NCHUNK):
        slot = k % 2
        cg, cw = pend
        cg.wait()
        cw.wait()
        if k + 1 < NCHUNK:
            pend = start(k + 1)
        gbuf, wbuf = gb[slot], wb[slot]

        @plsc.parallel_loop(0, VPC, 1, unroll=UNROLL)
        def _(j, gbuf=gbuf, wbuf=wbuf):
            off = j * L
            g = gbuf[pl.ds(off, L)]
            wv = wbuf[pl.ds(off, L)]
            kds = _kds(g)
            bin1 = (kds >> 16) + 32768
            w2 = jnp.where(bin1 == b1v, wv, 0.0)
            bin2 = kds & 65535
            plsc.addupdate_scatter(hist, [bin2 >> 7, bin2 & 127], w2)

    pltpu.sync_copy(hist, h2_hbm.at[wid])


@functools.cache
def _hist2():
    return pl.kernel(
        _hist2_body,
        out_type=jax.ShapeDtypeStruct((NW, HROWS, 128), jnp.float32),
        mesh=_mesh(),
        scratch_types=[
            pltpu.VMEM((CHUNK,), jnp.float32),
            pltpu.VMEM((CHUNK,), jnp.float32),
            pltpu.VMEM((CHUNK,), jnp.float32),
            pltpu.VMEM((CHUNK,), jnp.float32),
            pltpu.VMEM((HROWS, 128), jnp.float32),
            pltpu.VMEM((L,), jnp.int32),
            pltpu.SemaphoreType.DMA,
            pltpu.SemaphoreType.DMA,
        ],
        compiler_params=pltpu.CompilerParams(needs_layout_passes=False),
    )


def _final_body(oi1_ref, of1_ref, h2_ref, eps_ref, g_ref, w_ref, o_ref,
                carry, gstar_s, imin_s, R_s, G_s):
    pid = pl.program_id(0)
    eps = eps_ref[0, 0]

    @pl.when(pid == 0)
    def _():
        b1 = oi1_ref[0, 0]
        adj = oi1_ref[0, 2]
        r1 = of1_ref[0, 0]
        H = jnp.sum(h2_ref[...], axis=0)
        bins = _bin2d(HROWS)
        Hadj = H + jnp.where(bins == adj, eps, 0.0)
        cum = _cumsum_rowmajor(Hadj)
        b2a = jnp.min(jnp.where(cum >= r1, bins, BIG))
        lastnz = jnp.max(jnp.where(Hadj > 0, bins, jnp.int32(-1)))
        b2 = jnp.minimum(b2a, jnp.maximum(lastnz, 0))
        G = jnp.sum(jnp.where(bins == b2, Hadj, 0.0))
        e2 = jnp.sum(jnp.where(bins == b2, cum, 0.0)) - G
        R = jnp.maximum(jnp.minimum(r1 - e2, G), 0.0)
        gstar_s[0] = ((b1 - 32768) << 16) | b2
        imin_s[0] = oi1_ref[0, 1]
        R_s[0] = R
        G_s[0] = G
        carry[0] = 0.0

    gstar = gstar_s[0]
    imin = imin_s[0]
    R = R_s[0]
    G = G_s[0]

    g = g_ref[...]
    w = w_ref[...]
    kds = _kds(g)
    blk_lo = pid * (BR * 128)
    has_imin = jnp.logical_and(imin >= blk_lo, imin < blk_lo + BR * 128)
    m = kds == gstar
    slow = jnp.logical_or(jnp.any(m), has_imin)

    @pl.when(jnp.logical_not(slow))
    def _():
        o_ref[...] = jnp.where(kds < gstar, 0.0, w)

    @pl.when(slow)
    def _():
        gi = blk_lo + _bin2d(BR)
        W = jnp.where(gi == imin, w + eps, w)
        mw = jnp.where(m, W, 0.0)
        F = carry[0] + _cumsum_rowmajor(mw)
        sub = jnp.clip(R - (G - F), 0.0, W)
        o_ref[...] = jnp.where(kds < gstar, 0.0,
                               jnp.where(kds > gstar, W, W - sub))
        carry[0] = carry[0] + jnp.sum(mw)


_final = pl.pallas_call(
    _final_body,
    grid=(NBLK,),
    in_specs=[
        pl.BlockSpec((8, 128), lambda i: (0, 0)),
        pl.BlockSpec((8, 128), lambda i: (0, 0)),
        pl.BlockSpec((NW, HROWS, 128), lambda i: (0, 0, 0)),
        pl.BlockSpec(memory_space=pltpu.SMEM),
        pl.BlockSpec((BR, 128), lambda i: (i, 0)),
        pl.BlockSpec((BR, 128), lambda i: (i, 0)),
    ],
    out_specs=pl.BlockSpec((BR, 128), lambda i: (i, 0)),
    out_shape=jax.ShapeDtypeStruct((NROWS, 128), jnp.float32),
    scratch_shapes=[
        pltpu.SMEM((1,), jnp.float32),
        pltpu.SMEM((1,), jnp.int32),
        pltpu.SMEM((1,), jnp.int32),
        pltpu.SMEM((1,), jnp.float32),
        pltpu.SMEM((1,), jnp.float32),
    ],
)


def kernel(weights, grad, epsilon):
    eps = jnp.float32(epsilon)
    eps_arr = jnp.reshape(eps, (1, 1))
    zeros_bins = jnp.zeros((HROWS, 128), jnp.float32)

    h1, mv, mi = _hist1()(grad, weights, zeros_bins)
    oi1, of1, b1row = _scan1(h1, mv, mi, eps_arr)

    h2 = _hist2()(grad, weights, zeros_bins, b1row)
    out2 = _final(oi1, of1, h2, eps_arr,
                  grad.reshape(NROWS, 128), weights.reshape(NROWS, 128))
    return out2.reshape(N)


# speedup vs baseline: 1.0206x; 1.0206x over previous
import functools

import jax
import jax.numpy as jnp
from jax import lax
from jax.experimental import pallas as pl
from jax.experimental.pallas import tpu as pltpu
from jax.experimental.pallas import tpu_sc as plsc

N = 4194304
NC, NS, L = 2, 16, 16
NW = NC * NS
SHARD = N // NW
CHUNK = 8192
NCHUNK = SHARD // CHUNK
VPC = CHUNK // L
NBIN = 65536
HROWS = NBIN // 128
BR = 2048
NROWS = N // 128
NBLK = NROWS // BR
BIG = 2147483647
SIGN = -2147483648

@functools.cache
def _mesh():
    return plsc.VectorSubcoreMesh(
        core_axis_name="c", subcore_axis_name="s", num_cores=NC, num_subcores=NS
    )


def _kds(g):
    bits = lax.bitcast_convert_type(g, jnp.int32)
    return jnp.where(bits < 0, bits ^ SIGN, ~bits)


def _cumsum_axis1(x):
    for s in (1, 2, 4, 8, 16, 32, 64):
        x = x + jnp.concatenate(
            [jnp.zeros((x.shape[0], s), x.dtype), x[:, :-s]], axis=1)
    return x


def _cumsum_axis0(x):
    s = 1
    while s < x.shape[0]:
        x = x + jnp.concatenate(
            [jnp.zeros((s, x.shape[1]), x.dtype), x[:-s, :]], axis=0)
        s *= 2
    return x


def _cumsum_rowmajor(x):
    cl = _cumsum_axis1(x)
    rs = cl[:, 127:128]
    rp = _cumsum_axis0(rs) - rs
    return cl + rp


def _bin2d(rows):
    r = lax.broadcasted_iota(jnp.int32, (rows, 128), 0)
    c = lax.broadcasted_iota(jnp.int32, (rows, 128), 1)
    return r * 128 + c


UNROLL = 8


def _hist1_body(grad_hbm, w_hbm, zeros_hbm, h1_hbm, minv_hbm, mini_hbm,
                gbuf0, gbuf1, wbuf0, wbuf1, hist, vf, vi, sem0, sem1):
    cid = lax.axis_index("c")
    sid = lax.axis_index("s")
    wid = sid * NC + cid
    base = wid * SHARD
    gb = (gbuf0, gbuf1)
    wb = (wbuf0, wbuf1)
    sems = (sem0, sem1)
    zc = pltpu.async_copy(zeros_hbm, hist, sem0)

    def start(k):
        slot = k % 2
        cg = pltpu.async_copy(
            grad_hbm.at[pl.ds(base + k * CHUNK, CHUNK)], gb[slot], sems[slot])
        cw = pltpu.async_copy(
            w_hbm.at[pl.ds(base + k * CHUNK, CHUNK)], wb[slot], sems[slot])
        return cg, cw

    zc.wait()
    pend = start(0)
    lane = lax.iota(jnp.int32, L)
    mn = jnp.full((L,), jnp.inf, jnp.float32)
    mi = jnp.zeros((L,), jnp.int32)
    for k in range(NCHUNK):
        slot = k % 2
        cg, cw = pend
        cg.wait()
        cw.wait()
        if k + 1 < NCHUNK:
            pend = start(k + 1)
        gbuf, wbuf = gb[slot], wb[slot]
        cbase = base + k * CHUNK

        @plsc.parallel_loop(0, VPC, 1, unroll=UNROLL, carry=(mn, mi))
        def mnmi(j, c, gbuf=gbuf, wbuf=wbuf, cbase=cbase):
            mn0, mi0 = c
            off = j * L
            g = gbuf[pl.ds(off, L)]
            wv = wbuf[pl.ds(off, L)]
            kds = _kds(g)
            bin1 = (kds >> 16) + 32768
            plsc.addupdate_scatter(hist, [bin1 >> 7, bin1 & 127], wv)
            m = g < mn0
            gi = lane + (cbase + off)
            return (jnp.where(m, g, mn0), jnp.where(m, gi, mi0))

        mn, mi = mnmi

    vf[...] = mn
    vi[...] = mi
    pltpu.sync_copy(hist, h1_hbm.at[wid])
    pltpu.sync_copy(vf, minv_hbm.at[wid])
    pltpu.sync_copy(vi, mini_hbm.at[wid])


@functools.cache
def _hist1():
    return pl.kernel(
        _hist1_body,
        out_type=[
            jax.ShapeDtypeStruct((NW, HROWS, 128), jnp.float32),
            jax.ShapeDtypeStruct((NW, L), jnp.float32),
            jax.ShapeDtypeStruct((NW, L), jnp.int32),
        ],
        mesh=_mesh(),
        scratch_types=[
            pltpu.VMEM((CHUNK,), jnp.float32),
            pltpu.VMEM((CHUNK,), jnp.float32),
            pltpu.VMEM((CHUNK,), jnp.float32),
            pltpu.VMEM((CHUNK,), jnp.float32),
            pltpu.VMEM((HROWS, 128), jnp.float32),
            pltpu.VMEM((L,), jnp.float32),
            pltpu.VMEM((L,), jnp.int32),
            pltpu.SemaphoreType.DMA,
            pltpu.SemaphoreType.DMA,
        ],
        compiler_params=pltpu.CompilerParams(needs_layout_passes=False),
    )


def _scan1_body(h1_ref, minv_ref, mini_ref, eps_ref, oi_ref, of_ref, ob_ref):
    eps = eps_ref[0, 0]
    H = jnp.sum(h1_ref[...], axis=0)
    mv = minv_ref[...]
    gmin = jnp.min(mv)
    imin = jnp.min(jnp.where(mv == gmin, mini_ref[...], BIG))
    kmin = jnp.max(_kds(mv))
    hbmin = (kmin >> 16) + 32768
    bins = _bin2d(HROWS)
    Hadj = H + jnp.where(bins == hbmin, eps, 0.0)
    cum = _cumsum_rowmajor(Hadj)
    b1 = jnp.min(jnp.where(cum >= eps, bins, BIG))
    hb1 = jnp.sum(jnp.where(bins == b1, Hadj, 0.0))
    e1 = jnp.sum(jnp.where(bins == b1, cum, 0.0)) - hb1
    r1 = jnp.minimum(eps - e1, hb1)
    adj = jnp.where(hbmin == b1, kmin & 65535, jnp.int32(-1))
    oi = jnp.where(bins == 0, b1,
                   jnp.where(bins == 1, imin,
                             jnp.where(bins == 2, adj, 0)))
    oi_ref[...] = oi[:8, :]
    of_ref[...] = jnp.where(bins[:8, :] == 0, r1, 0.0)
    ob_ref[...] = jnp.zeros((NW, L), jnp.int32) + b1


_scan1 = pl.pallas_call(
    _scan1_body,
    in_specs=[
        pl.BlockSpec(memory_space=pltpu.VMEM),
        pl.BlockSpec(memory_space=pltpu.VMEM),
        pl.BlockSpec(memory_space=pltpu.VMEM),
        pl.BlockSpec(memory_space=pltpu.SMEM),
    ],
    out_specs=[
        pl.BlockSpec(memory_space=pltpu.VMEM),
        pl.BlockSpec(memory_space=pltpu.VMEM),
        pl.BlockSpec(memory_space=pltpu.VMEM),
    ],
    out_shape=[
        jax.ShapeDtypeStruct((8, 128), jnp.int32),
        jax.ShapeDtypeStruct((8, 128), jnp.float32),
        jax.ShapeDtypeStruct((NW, L), jnp.int32),
    ],
)


def _hist2_body(grad_hbm, w_hbm, zeros_hbm, b1_hbm, h2_hbm,
                gbuf0, gbuf1, wbuf0, wbuf1, hist, b1buf, sem0, sem1):
    cid = lax.axis_index("c")
    sid = lax.axis_index("s")
    wid = sid * NC + cid
    base = wid * SHARD
    gb = (gbuf0, gbuf1)
    wb = (wbuf0, wbuf1)
    sems = (sem0, sem1)
    pltpu.sync_copy(zeros_hbm, hist)
    pltpu.sync_copy(b1_hbm.at[0], b1buf)
    b1v = b1buf[...]

    def start(k):
        slot = k % 2
        cg = pltpu.async_copy(
            grad_hbm.at[pl.ds(base + k * CHUNK, CHUNK)], gb[slot], sems[slot])
        cw = pltpu.async_copy(
            w_hbm.at[pl.ds(base + k * CHUNK, CHUNK)], wb[slot], sems[slot])
        return cg, cw

    pend = start(0)
    for k in range(NCHUNK):
        slot = k % 2
        cg, cw = pend
        cg.wait()
        cw.wait()
        if k + 1 < NCHUNK:
            pend = start(k + 1)
        gbuf, wbuf = gb[slot], wb[slot]

        @plsc.parallel_loop(0, VPC, 1, unroll=UNROLL)
        def _(j, gbuf=gbuf, wbuf=wbuf):
            off = j * L
            g = gbuf[pl.ds(off, L)]
            wv = wbuf[pl.ds(off, L)]
            kds = _kds(g)
            bin1 = (kds >> 16) + 32768
            w2 = jnp.where(bin1 == b1v, wv, 0.0)
            bin2 = kds & 65535
            plsc.addupdate_scatter(hist, [bin2 >> 7, bin2 & 127], w2)

    pltpu.sync_copy(hist, h2_hbm.at[wid])


@functools.cache
def _hist2():
    return pl.kernel(
        _hist2_body,
        out_type=jax.ShapeDtypeStruct((NW, HROWS, 128), jnp.float32),
        mesh=_mesh(),
        scratch_types=[
            pltpu.VMEM((CHUNK,), jnp.float32),
            pltpu.VMEM((CHUNK,), jnp.float32),
            pltpu.VMEM((CHUNK,), jnp.float32),
            pltpu.VMEM((CHUNK,), jnp.float32),
            pltpu.VMEM((HROWS, 128), jnp.float32),
            pltpu.VMEM((L,), jnp.int32),
            pltpu.SemaphoreType.DMA,
            pltpu.SemaphoreType.DMA,
        ],
        compiler_params=pltpu.CompilerParams(needs_layout_passes=False),
    )


def _final_body(oi1_ref, of1_ref, h2_ref, eps_ref, g_ref, w_ref, o_ref,
                carry, gstar_s, imin_s, R_s, G_s):
    pid = pl.program_id(0)
    eps = eps_ref[0, 0]

    @pl.when(pid == 0)
    def _():
        b1 = oi1_ref[0, 0]
        adj = oi1_ref[0, 2]
        r1 = of1_ref[0, 0]
        H = jnp.sum(h2_ref[...], axis=0)
        bins = _bin2d(HROWS)
        Hadj = H + jnp.where(bins == adj, eps, 0.0)
        cum = _cumsum_rowmajor(Hadj)
        b2a = jnp.min(jnp.where(cum >= r1, bins, BIG))
        lastnz = jnp.max(jnp.where(Hadj > 0, bins, jnp.int32(-1)))
        b2 = jnp.minimum(b2a, jnp.maximum(lastnz, 0))
        G = jnp.sum(jnp.where(bins == b2, Hadj, 0.0))
        e2 = jnp.sum(jnp.where(bins == b2, cum, 0.0)) - G
        R = jnp.maximum(jnp.minimum(r1 - e2, G), 0.0)
        gstar_s[0] = ((b1 - 32768) << 16) | b2
        imin_s[0] = oi1_ref[0, 1]
        R_s[0] = R
        G_s[0] = G
        carry[0] = 0.0

    gstar = gstar_s[0]
    imin = imin_s[0]
    R = R_s[0]
    G = G_s[0]

    g = g_ref[...]
    w = w_ref[...]
    kds = _kds(g)
    blk_lo = pid * (BR * 128)
    has_imin = jnp.logical_and(imin >= blk_lo, imin < blk_lo + BR * 128)
    m = kds == gstar
    slow = jnp.logical_or(jnp.any(m), has_imin)

    @pl.when(jnp.logical_not(slow))
    def _():
        o_ref[...] = jnp.where(kds < gstar, 0.0, w)

    @pl.when(slow)
    def _():
        gi = blk_lo + _bin2d(BR)
        W = jnp.where(gi == imin, w + eps, w)
        mw = jnp.where(m, W, 0.0)
        F = carry[0] + _cumsum_rowmajor(mw)
        sub = jnp.clip(R - (G - F), 0.0, W)
        o_ref[...] = jnp.where(kds < gstar, 0.0,
                               jnp.where(kds > gstar, W, W - sub))
        carry[0] = carry[0] + jnp.sum(mw)


_final = pl.pallas_call(
    _final_body,
    grid=(NBLK,),
    in_specs=[
        pl.BlockSpec((8, 128), lambda i: (0, 0)),
        pl.BlockSpec((8, 128), lambda i: (0, 0)),
        pl.BlockSpec((NW, HROWS, 128), lambda i: (0, 0, 0)),
        pl.BlockSpec(memory_space=pltpu.SMEM),
        pl.BlockSpec((BR, 128), lambda i: (i, 0)),
        pl.BlockSpec((BR, 128), lambda i: (i, 0)),
    ],
    out_specs=pl.BlockSpec((BR, 128), lambda i: (i, 0)),
    out_shape=jax.ShapeDtypeStruct((NROWS, 128), jnp.float32),
    scratch_shapes=[
        pltpu.SMEM((1,), jnp.float32),
        pltpu.SMEM((1,), jnp.int32),
        pltpu.SMEM((1,), jnp.int32),
        pltpu.SMEM((1,), jnp.float32),
        pltpu.SMEM((1,), jnp.float32),
    ],
)


def kernel(weights, grad, epsilon):
    eps = jnp.float32(epsilon)
    eps_arr = jnp.reshape(eps, (1, 1))
    zeros_bins = jnp.zeros((HROWS, 128), jnp.float32)

    h1, mv, mi = _hist1()(grad, weights, zeros_bins)
    oi1, of1, b1row = _scan1(h1, mv, mi, eps_arr)

    h2 = _hist2()(grad, weights, zeros_bins, b1row)
    out2 = _final(oi1, of1, h2, eps_arr,
                  grad.reshape(NROWS, 128), weights.reshape(NROWS, 128))
    return out2.reshape(N)
